# baseline (speedup 1.0000x reference)
import jax
import jax.numpy as jnp
from jax.experimental import pallas as pl
from jax.experimental.pallas import tpu as pltpu

_NUM_MASKS = 512
_N_POINTS = 20000
_ROW_BLOCK = 64
_GRID = _NUM_MASKS // _ROW_BLOCK
_ALPHA = 0.25


def _loss_kernel(nb_ref, x_ref, t_ref, out_ref, acc_ref):
    step = pl.program_id(0)

    @pl.when(step == 0)
    def _init():
        acc_ref[0] = 0.0
        acc_ref[1] = 0.0

    x = x_ref[...]
    t = t_ref[...]

    u = jnp.exp(-jnp.abs(x))
    r = 1.0 / (1.0 + u)
    p = jnp.where(x >= 0.0, r, u * r)
    log1p_u = jnp.log1p(u)

    ce = jnp.maximum(x, 0.0) - x * t + log1p_u
    pt = p * t + (1.0 - p) * (1.0 - t)
    ompt = 1.0 - pt
    alpha_t = _ALPHA * t + (1.0 - _ALPHA) * (1.0 - t)
    focal_el = alpha_t * ce * (ompt * ompt)

    s_pt = jnp.sum(p * t, axis=1)
    s_p = jnp.sum(p, axis=1)
    s_t = jnp.sum(t, axis=1)
    s_f = jnp.sum(focal_el)

    dice_rows = 1.0 - (2.0 * s_pt + 1.0) / (s_p + s_t + 1.0)
    acc_ref[0] += jnp.sum(dice_rows)
    acc_ref[1] += s_f

    @pl.when(step == _GRID - 1)
    def _finish():
        inv_nb = 1.0 / (nb_ref[0] + 1e-06)
        dice = acc_ref[0] * inv_nb
        focal = acc_ref[1] * (inv_nb / _N_POINTS)
        out_ref[0] = dice + focal
        out_ref[1] = dice
        out_ref[2] = focal


def kernel(mask_logits_pred, inst_mask_gt, num_boxes):
    nb = jnp.asarray(num_boxes, dtype=jnp.float32).reshape((1,))
    out = pl.pallas_call(
        _loss_kernel,
        grid=(_GRID,),
        in_specs=[
            pl.BlockSpec(memory_space=pltpu.SMEM),
            pl.BlockSpec((_ROW_BLOCK, _N_POINTS), lambda i: (i, 0)),
            pl.BlockSpec((_ROW_BLOCK, _N_POINTS), lambda i: (i, 0)),
        ],
        out_specs=pl.BlockSpec(memory_space=pltpu.SMEM),
        out_shape=jax.ShapeDtypeStruct((3,), jnp.float32),
        scratch_shapes=[pltpu.SMEM((2,), jnp.float32)],
    )(nb, mask_logits_pred, inst_mask_gt)
    return (out[0], out[1], out[2])


# speedup vs baseline: 1.1161x; 1.1161x over previous
import jax
import jax.numpy as jnp
from jax.experimental import pallas as pl
from jax.experimental.pallas import tpu as pltpu

_NUM_MASKS = 512
_N_POINTS = 20000
_ROW_BLOCK = 8
_GRID = _NUM_MASKS // _ROW_BLOCK
_CH = 512
_NFULL = _N_POINTS // _CH
_REM = _N_POINTS - _NFULL * _CH
_ALPHA = 0.25


def _elementwise(x, t):
    u = jnp.exp(-jnp.abs(x))
    w = 1.0 + u
    r = 1.0 / w
    ur = u * r
    p = jnp.where(x >= 0.0, r, ur)
    log1p_u = jnp.log(w)
    ce = jnp.maximum(x, 0.0) - x * t + log1p_u
    den_v = p + t
    ptv = p * t
    ompt = den_v - (ptv + ptv)
    alpha_t = (1.0 - _ALPHA) - (1.0 - 2.0 * _ALPHA) * t
    focal_el = alpha_t * ce * (ompt * ompt)
    return focal_el, den_v, ptv


def _loss_kernel(nb_ref, x_ref, t_ref, out_ref, acc_ref):
    step = pl.program_id(0)

    @pl.when(step == 0)
    def _init():
        acc_ref[0] = 0.0
        acc_ref[1] = 0.0

    acc_f = jnp.zeros((_ROW_BLOCK, _CH), jnp.float32)
    acc_den = jnp.zeros((_ROW_BLOCK, _CH), jnp.float32)
    acc_pt = jnp.zeros((_ROW_BLOCK, _CH), jnp.float32)
    for j in range(_NFULL):
        x = x_ref[:, j * _CH:(j + 1) * _CH]
        t = t_ref[:, j * _CH:(j + 1) * _CH]
        f_v, den_v, ptv = _elementwise(x, t)
        acc_f = acc_f + f_v
        acc_den = acc_den + den_v
        acc_pt = acc_pt + ptv

    xr = x_ref[:, _NFULL * _CH:]
    tr = t_ref[:, _NFULL * _CH:]
    f_r, den_r, pt_r = _elementwise(xr, tr)

    s_pt = jnp.sum(acc_pt, axis=1) + jnp.sum(pt_r, axis=1)
    s_den = jnp.sum(acc_den, axis=1) + jnp.sum(den_r, axis=1)
    f_step = jnp.sum(acc_f) + jnp.sum(f_r)

    dice_rows = 1.0 - (2.0 * s_pt + 1.0) / (s_den + 1.0)
    acc_ref[0] += jnp.sum(dice_rows)
    acc_ref[1] += f_step

    @pl.when(step == _GRID - 1)
    def _finish():
        inv_nb = 1.0 / (nb_ref[0] + 1e-06)
        dice = acc_ref[0] * inv_nb
        focal = acc_ref[1] * (inv_nb / _N_POINTS)
        out_ref[0] = dice + focal
        out_ref[1] = dice
        out_ref[2] = focal


def kernel(mask_logits_pred, inst_mask_gt, num_boxes):
    nb = jnp.asarray(num_boxes, dtype=jnp.float32).reshape((1,))
    out = pl.pallas_call(
        _loss_kernel,
        grid=(_GRID,),
        in_specs=[
            pl.BlockSpec(memory_space=pltpu.SMEM),
            pl.BlockSpec((_ROW_BLOCK, _N_POINTS), lambda i: (i, 0)),
            pl.BlockSpec((_ROW_BLOCK, _N_POINTS), lambda i: (i, 0)),
        ],
        out_specs=pl.BlockSpec(memory_space=pltpu.SMEM),
        out_shape=jax.ShapeDtypeStruct((3,), jnp.float32),
        scratch_shapes=[pltpu.SMEM((2,), jnp.float32)],
    )(nb, mask_logits_pred, inst_mask_gt)
    return (out[0], out[1], out[2])


# speedup vs baseline: 1.3414x; 1.2019x over previous
import jax
import jax.numpy as jnp
from jax.experimental import pallas as pl
from jax.experimental.pallas import tpu as pltpu

_NUM_MASKS = 512
_N_POINTS = 20000
_ROW_BLOCK = 32
_RG = 8
_NRG = _ROW_BLOCK // _RG
_GRID = _NUM_MASKS // _ROW_BLOCK
_CH = 512
_NFULL = _N_POINTS // _CH
_REM = _N_POINTS - _NFULL * _CH
_ALPHA = 0.25


def _elementwise(x, t):
    u = jnp.exp(-jnp.abs(x))
    w = 1.0 + u
    r = 1.0 / w
    ur = u * r
    p = jnp.where(x >= 0.0, r, ur)
    log1p_u = jnp.log(w)
    ce = jnp.maximum(x, 0.0) - x * t + log1p_u
    den_v = p + t
    ptv = p * t
    ompt = den_v - (ptv + ptv)
    alpha_t = (1.0 - _ALPHA) - (1.0 - 2.0 * _ALPHA) * t
    focal_el = alpha_t * ce * (ompt * ompt)
    return focal_el, den_v, ptv


def _loss_kernel(nb_ref, x_ref, t_ref, out_ref, acc_ref):
    step = pl.program_id(0)

    @pl.when(step == 0)
    def _init():
        acc_ref[0] = 0.0
        acc_ref[1] = 0.0

    dice_step = 0.0
    f_step = 0.0
    for r in range(_NRG):
        r0, r1 = r * _RG, (r + 1) * _RG
        acc_f = jnp.zeros((_RG, _CH), jnp.float32)
        acc_den = jnp.zeros((_RG, _CH), jnp.float32)
        acc_pt = jnp.zeros((_RG, _CH), jnp.float32)
        for j in range(_NFULL):
            x = x_ref[r0:r1, j * _CH:(j + 1) * _CH]
            t = t_ref[r0:r1, j * _CH:(j + 1) * _CH]
            f_v, den_v, ptv = _elementwise(x, t)
            acc_f = acc_f + f_v
            acc_den = acc_den + den_v
            acc_pt = acc_pt + ptv

        xr = x_ref[r0:r1, _NFULL * _CH:]
        tr = t_ref[r0:r1, _NFULL * _CH:]
        f_r, den_r, pt_r = _elementwise(xr, tr)

        s_pt = jnp.sum(acc_pt, axis=1) + jnp.sum(pt_r, axis=1)
        s_den = jnp.sum(acc_den, axis=1) + jnp.sum(den_r, axis=1)
        dice_rows = 1.0 - (2.0 * s_pt + 1.0) / (s_den + 1.0)
        dice_step += jnp.sum(dice_rows)
        f_step += jnp.sum(acc_f) + jnp.sum(f_r)

    acc_ref[0] += dice_step
    acc_ref[1] += f_step

    @pl.when(step == _GRID - 1)
    def _finish():
        inv_nb = 1.0 / (nb_ref[0] + 1e-06)
        dice = acc_ref[0] * inv_nb
        focal = acc_ref[1] * (inv_nb / _N_POINTS)
        out_ref[0] = dice + focal
        out_ref[1] = dice
        out_ref[2] = focal


def kernel(mask_logits_pred, inst_mask_gt, num_boxes):
    nb = jnp.asarray(num_boxes, dtype=jnp.float32).reshape((1,))
    out = pl.pallas_call(
        _loss_kernel,
        grid=(_GRID,),
        in_specs=[
            pl.BlockSpec(memory_space=pltpu.SMEM),
            pl.BlockSpec((_ROW_BLOCK, _N_POINTS), lambda i: (i, 0)),
            pl.BlockSpec((_ROW_BLOCK, _N_POINTS), lambda i: (i, 0)),
        ],
        out_specs=pl.BlockSpec(memory_space=pltpu.SMEM),
        out_shape=jax.ShapeDtypeStruct((3,), jnp.float32),
        scratch_shapes=[pltpu.SMEM((2,), jnp.float32)],
    )(nb, mask_logits_pred, inst_mask_gt)
    return (out[0], out[1], out[2])
